# Initial kernel scaffold; baseline (speedup 1.0000x reference)
import jax
import jax.numpy as jnp
from jax.experimental import pallas as pl


def kernel(input, embedding, W_rel, W_self, W_atom, bias, unit_type, edge_index, edge_relation, node2graph):
    raise NotImplementedError("write your pallas kernel here")


# speedup vs baseline: 6.3944x; 6.3944x over previous
import functools

import jax
import jax.numpy as jnp
from jax import lax
from jax.experimental import pallas as pl
from jax.experimental.pallas import tpu as pltpu
import jax.experimental.pallas.tpu_sc as plsc

N = 10000
E = 320000
D = 128
NUM_UNIT = 512
UNIT_BOUNDARY = 256
NUM_REL = 4
NUM_BLOCKS = 3
B = 64

NP = 10240
BL = 1024
NPB = NP // BL
NC = 2
NS = 16
TILES = NC * NS
K = 128
CPT = 80
EPT = CPT * K
EP = TILES * EPT
ACC_R = NP + 256
ZCH = 82

_HIGH = jax.lax.Precision.HIGHEST


def _dot(a, b):
    return jnp.dot(a, b, precision=_HIGH, preferred_element_type=jnp.float32)


_mesh = plsc.VectorSubcoreMesh(core_axis_name="c", subcore_axis_name="s")


def _prep_body(src_h, dst_h, rel_h, ut_h, ga_h, da_h, gr_h, dr_h,
               ut_v, sv, dv, rv, gav, dav, grv, drv):
    c = lax.axis_index("c")
    s = lax.axis_index("s")
    wid = s * NC + c
    trash = NP + s
    pltpu.sync_copy(ut_h, ut_v)
    CH = 2048
    lanes = lax.broadcasted_iota(jnp.int32, (16,), 0)

    def chunk(ch, _):
        base = wid * EPT + ch * CH
        pltpu.sync_copy(src_h.at[pl.ds(base, CH)], sv)
        pltpu.sync_copy(dst_h.at[pl.ds(base, CH)], dv)
        pltpu.sync_copy(rel_h.at[pl.ds(base, CH)], rv)

        def grp(j, _):
            s16 = sv[pl.ds(j * 16, 16)]
            d16 = dv[pl.ds(j * 16, 16)]
            r16 = rv[pl.ds(j * 16, 16)]
            st = plsc.load_gather(ut_v, [s16])
            dt = plsc.load_gather(ut_v, [d16])
            sa = st >= UNIT_BOUNDARY
            da = dt >= UNIT_BOUNDARY
            inr = (base + j * 16 + lanes) < E
            cross = sa & (~da) & inr
            mono = (~sa) & (~da) & inr
            gav[pl.ds(j * 16, 16)] = s16
            dav[pl.ds(j * 16, 16)] = jnp.where(cross, d16, trash)
            grv[pl.ds(j * 16, 16)] = r16 * NP + s16
            drv[pl.ds(j * 16, 16)] = jnp.where(mono, d16, trash)
            return 0

        lax.fori_loop(0, CH // 16, grp, 0)
        pltpu.sync_copy(gav, ga_h.at[pl.ds(base, CH)])
        pltpu.sync_copy(dav, da_h.at[pl.ds(base, CH)])
        pltpu.sync_copy(grv, gr_h.at[pl.ds(base, CH)])
        pltpu.sync_copy(drv, dr_h.at[pl.ds(base, CH)])
        return 0

    lax.fori_loop(0, EPT // CH, chunk, 0)


def _sc_prep(src, dst, rel, ut):
    ei = jax.ShapeDtypeStruct((EP,), jnp.int32)
    fn = pl.kernel(
        _prep_body,
        out_type=(ei, ei, ei, ei),
        mesh=_mesh,
        compiler_params=pltpu.CompilerParams(needs_layout_passes=False),
        scratch_types=[
            pltpu.VMEM((NP,), jnp.int32),
            pltpu.VMEM((2048,), jnp.int32),
            pltpu.VMEM((2048,), jnp.int32),
            pltpu.VMEM((2048,), jnp.int32),
            pltpu.VMEM((2048,), jnp.int32),
            pltpu.VMEM((2048,), jnp.int32),
            pltpu.VMEM((2048,), jnp.int32),
            pltpu.VMEM((2048,), jnp.int32),
        ],
    )
    return fn(src, dst, rel, ut)


def _scatter_body(table_h, g_h, d_h, out_h, gv, dv, rows_v, zv, acc, sem):
    c = lax.axis_index("c")
    s = lax.axis_index("s")
    wid = s * NC + c

    def zlane(t, _):
        zv[t // 8, pl.ds((t % 8) * 16, 16)] = jnp.zeros((16,), jnp.float32)
        return 0

    lax.fori_loop(0, ZCH * 8, zlane, 0)

    def zcopy(k, _):
        pltpu.sync_copy(zv, acc.at[pl.ds(s * (8 * ZCH) + k * ZCH, ZCH), :])
        return 0

    lax.fori_loop(0, 8, zcopy, 0)
    plsc.subcore_barrier()

    def chunk(k, _):
        base = wid * EPT + k * K
        pltpu.sync_copy(g_h.at[pl.ds(base, K)], gv)
        pltpu.sync_copy(d_h.at[pl.ds(base, K)], dv)
        pltpu.async_copy(table_h.at[gv], rows_v, sem).wait()
        pltpu.sync_copy(rows_v, acc.at[dv], add=True)
        return 0

    lax.fori_loop(0, CPT, chunk, 0)
    plsc.subcore_barrier()

    def ocopy(k, _):
        r0 = s * (NP // NS) + k * 128
        pltpu.sync_copy(acc.at[pl.ds(r0, 128), :], out_h.at[c, pl.ds(r0, 128), :])
        return 0

    lax.fori_loop(0, NP // NS // 128, ocopy, 0)


def _sc_scatter(table, gidx, didx):
    fn = pl.kernel(
        _scatter_body,
        out_type=jax.ShapeDtypeStruct((NC, NP, D), jnp.float32),
        mesh=_mesh,
        scratch_types=[
            pltpu.VMEM((K,), jnp.int32),
            pltpu.VMEM((K,), jnp.int32),
            pltpu.VMEM((K, D), jnp.float32),
            pltpu.VMEM((ZCH, D), jnp.float32),
            pltpu.VMEM_SHARED((ACC_R, D), jnp.float32),
            pltpu.SemaphoreType.DMA,
        ],
    )
    return fn(table, gidx, didx)


def _k0_body(ut_ref, emb_ref, wrel_ref, x_ref, xw_ref):
    ut = ut_ref[0, 0, :]
    oh = (ut[:, None] ==
          lax.broadcasted_iota(jnp.int32, (BL, NUM_UNIT), 1)).astype(jnp.float32)
    x = _dot(oh, emb_ref[...])
    x_ref[...] = x
    for r in range(NUM_REL):
        xw_ref[r, :, :] = _dot(x, wrel_ref[r])


def _tc_k0(ut3, emb, wrel):
    return pl.pallas_call(
        _k0_body,
        grid=(NPB,),
        in_specs=[
            pl.BlockSpec((1, 1, BL), lambda i: (i, 0, 0)),
            pl.BlockSpec((NUM_UNIT, D), lambda i: (0, 0)),
            pl.BlockSpec((NUM_REL, D, D), lambda i: (0, 0, 0)),
        ],
        out_specs=[
            pl.BlockSpec((BL, D), lambda i: (i, 0)),
            pl.BlockSpec((NUM_REL, BL, D), lambda i: (0, i, 0)),
        ],
        out_shape=[
            jax.ShapeDtypeStruct((NP, D), jnp.float32),
            jax.ShapeDtypeStruct((NUM_REL, NP, D), jnp.float32),
        ],
    )(ut3, emb, wrel)


def _k1_body(agg_ref, atomp_ref, x_ref, ws_ref, wa_ref, b_ref, wrel_ref,
             li_ref, xw_ref, atom_ref):
    atom = atomp_ref[0] + atomp_ref[1]
    agg = agg_ref[0] + agg_ref[1]
    li = jnp.maximum(
        agg + _dot(x_ref[...], ws_ref[...]) + _dot(atom, wa_ref[...])
        + b_ref[...], 0.0)
    li_ref[...] = li
    atom_ref[...] = atom
    for r in range(NUM_REL):
        xw_ref[r, :, :] = _dot(li, wrel_ref[r])


def _tc_k1(agg, atomp, x, ws, wa, b, wrel):
    return pl.pallas_call(
        _k1_body,
        grid=(NPB,),
        in_specs=[
            pl.BlockSpec((NC, BL, D), lambda i: (0, i, 0)),
            pl.BlockSpec((NC, BL, D), lambda i: (0, i, 0)),
            pl.BlockSpec((BL, D), lambda i: (i, 0)),
            pl.BlockSpec((D, D), lambda i: (0, 0)),
            pl.BlockSpec((D, D), lambda i: (0, 0)),
            pl.BlockSpec((1, D), lambda i: (0, 0)),
            pl.BlockSpec((NUM_REL, D, D), lambda i: (0, 0, 0)),
        ],
        out_specs=[
            pl.BlockSpec((BL, D), lambda i: (i, 0)),
            pl.BlockSpec((NUM_REL, BL, D), lambda i: (0, i, 0)),
            pl.BlockSpec((BL, D), lambda i: (i, 0)),
        ],
        out_shape=[
            jax.ShapeDtypeStruct((NP, D), jnp.float32),
            jax.ShapeDtypeStruct((NUM_REL, NP, D), jnp.float32),
            jax.ShapeDtypeStruct((NP, D), jnp.float32),
        ],
    )(agg, atomp, x, ws, wa, b, wrel)


def _k2_body(agg_ref, li_ref_in, atom_ref, ws_ref, wa_ref, b_ref, wrel_ref,
             li_ref, xw_ref):
    agg = agg_ref[0] + agg_ref[1]
    li = jnp.maximum(
        agg + _dot(li_ref_in[...], ws_ref[...]) + _dot(atom_ref[...], wa_ref[...])
        + b_ref[...], 0.0)
    li_ref[...] = li
    for r in range(NUM_REL):
        xw_ref[r, :, :] = _dot(li, wrel_ref[r])


def _tc_k2(agg, li_prev, atom, ws, wa, b, wrel):
    return pl.pallas_call(
        _k2_body,
        grid=(NPB,),
        in_specs=[
            pl.BlockSpec((NC, BL, D), lambda i: (0, i, 0)),
            pl.BlockSpec((BL, D), lambda i: (i, 0)),
            pl.BlockSpec((BL, D), lambda i: (i, 0)),
            pl.BlockSpec((D, D), lambda i: (0, 0)),
            pl.BlockSpec((D, D), lambda i: (0, 0)),
            pl.BlockSpec((1, D), lambda i: (0, 0)),
            pl.BlockSpec((NUM_REL, D, D), lambda i: (0, 0, 0)),
        ],
        out_specs=[
            pl.BlockSpec((BL, D), lambda i: (i, 0)),
            pl.BlockSpec((NUM_REL, BL, D), lambda i: (0, i, 0)),
        ],
        out_shape=[
            jax.ShapeDtypeStruct((NP, D), jnp.float32),
            jax.ShapeDtypeStruct((NUM_REL, NP, D), jnp.float32),
        ],
    )(agg, li_prev, atom, ws, wa, b, wrel)


def _k3_body(agg_ref, li_ref_in, atom_ref, ws_ref, wa_ref, b_ref,
             ut_ref, n2g_ref, nf_ref, gf_ref):
    agg = agg_ref[0] + agg_ref[1]
    li = jnp.maximum(
        agg + _dot(li_ref_in[...], ws_ref[...]) + _dot(atom_ref[...], wa_ref[...])
        + b_ref[...], 0.0)
    mono = (ut_ref[0, 0, :] < UNIT_BOUNDARY).astype(jnp.float32)
    nf = li * mono[:, None]
    nf_ref[...] = nf
    seg = n2g_ref[0, 0, :]
    oh = (lax.broadcasted_iota(jnp.int32, (B, BL), 0) ==
          seg[None, :]).astype(jnp.float32)

    @pl.when(pl.program_id(0) == 0)
    def _():
        gf_ref[...] = jnp.zeros((B, D), jnp.float32)

    gf_ref[...] += _dot(oh, nf)


def _tc_k3(agg, li_prev, atom, ws, wa, b, ut3, n2g3):
    return pl.pallas_call(
        _k3_body,
        grid=(NPB,),
        in_specs=[
            pl.BlockSpec((NC, BL, D), lambda i: (0, i, 0)),
            pl.BlockSpec((BL, D), lambda i: (i, 0)),
            pl.BlockSpec((BL, D), lambda i: (i, 0)),
            pl.BlockSpec((D, D), lambda i: (0, 0)),
            pl.BlockSpec((D, D), lambda i: (0, 0)),
            pl.BlockSpec((1, D), lambda i: (0, 0)),
            pl.BlockSpec((1, 1, BL), lambda i: (i, 0, 0)),
            pl.BlockSpec((1, 1, BL), lambda i: (i, 0, 0)),
        ],
        out_specs=[
            pl.BlockSpec((BL, D), lambda i: (i, 0)),
            pl.BlockSpec((B, D), lambda i: (0, 0)),
        ],
        out_shape=[
            jax.ShapeDtypeStruct((NP, D), jnp.float32),
            jax.ShapeDtypeStruct((B, D), jnp.float32),
        ],
        compiler_params=pltpu.CompilerParams(
            dimension_semantics=("arbitrary",)),
    )(agg, li_prev, atom, ws, wa, b, ut3, n2g3)


def kernel(input, embedding, W_rel, W_self, W_atom, bias, unit_type,
           edge_index, edge_relation, node2graph):
    del input
    i32 = jnp.int32
    ut = unit_type.astype(i32)
    utp = jnp.concatenate([ut, jnp.full((NP - N,), NUM_UNIT - 1, i32)])
    ut3 = utp.reshape(NPB, 1, BL)
    src = jnp.concatenate([edge_index[0].astype(i32),
                           jnp.zeros((EP - E,), i32)])
    dst = jnp.concatenate([edge_index[1].astype(i32),
                           jnp.zeros((EP - E,), i32)])
    rel = jnp.concatenate([edge_relation.astype(i32),
                           jnp.zeros((EP - E,), i32)])
    n2g = jnp.concatenate([node2graph.astype(i32), jnp.zeros((NP - N,), i32)])
    n2g3 = n2g.reshape(NPB, 1, BL)
    bias2 = bias.reshape(NUM_BLOCKS, 1, D)

    x, xw0 = _tc_k0(ut3, embedding, W_rel[0])
    ga, da, gr, dr = _sc_prep(src, dst, rel, utp)
    atom_parts = _sc_scatter(x, ga, da)
    agg0 = _sc_scatter(xw0.reshape(NUM_REL * NP, D), gr, dr)
    li1, xw1, atom = _tc_k1(agg0, atom_parts, x, W_self[0], W_atom[0],
                            bias2[0], W_rel[1])
    agg1 = _sc_scatter(xw1.reshape(NUM_REL * NP, D), gr, dr)
    li2, xw2 = _tc_k2(agg1, li1, atom, W_self[1], W_atom[1], bias2[1],
                      W_rel[2])
    agg2 = _sc_scatter(xw2.reshape(NUM_REL * NP, D), gr, dr)
    nf, gf = _tc_k3(agg2, li2, atom, W_self[2], W_atom[2], bias2[2],
                    ut3, n2g3)
    return gf, nf[:N]


# speedup vs baseline: 17.2071x; 2.6909x over previous
import jax
import jax.numpy as jnp
from jax import lax
from jax.experimental import pallas as pl
from jax.experimental.pallas import tpu as pltpu
import jax.experimental.pallas.tpu_sc as plsc

N = 10000
E = 320000
D = 128
NUM_UNIT = 512
UNIT_BOUNDARY = 256
NUM_REL = 4
NUM_BLOCKS = 3
B = 64

NP = 10240
BL = 1024
NPB = NP // BL
NC = 2
NS = 16
TILES = NC * NS
K = 128
CPT = 80
EPT = CPT * K
EP = TILES * EPT
HALF = NP // 2
ACC_R = HALF + NS
CAP = EPT + 256
CH = 2048

_HIGH = jax.lax.Precision.HIGHEST


def _dot(a, b):
    return jnp.dot(a, b, precision=_HIGH, preferred_element_type=jnp.float32)


_mesh = plsc.VectorSubcoreMesh(core_axis_name="c", subcore_axis_name="s")


def _prep_body(src_h, dst_h, rel_h, ut_h,
               ga_h, da_h, gr_h, dr_h, ca_h, cr_h,
               ut_v, sv, dv, rv, cntv,
               ga0_v, ga1_v, da0_v, da1_v, gr0_v, gr1_v, dr0_v, dr1_v):
    c = lax.axis_index("c")
    s = lax.axis_index("s")
    t = s * NC + c
    trash = jnp.int32(HALF + (t // NC))
    pltpu.sync_copy(ut_h, ut_v)
    lanes = lax.broadcasted_iota(jnp.int32, (16,), 0)
    zeros16 = jnp.zeros((16,), jnp.int32)
    trash16 = jnp.full((16,), trash, jnp.int32)

    def chunk(ch, offs):
        base = t * EPT + ch * CH
        pltpu.sync_copy(src_h.at[pl.ds(base, CH)], sv)
        pltpu.sync_copy(dst_h.at[pl.ds(base, CH)], dv)
        pltpu.sync_copy(rel_h.at[pl.ds(base, CH)], rv)

        def grp(j, offs):
            oa0, oa1, or0, or1 = offs
            s16 = sv[pl.ds(j * 16, 16)]
            d16 = dv[pl.ds(j * 16, 16)]
            r16 = rv[pl.ds(j * 16, 16)]
            st = plsc.load_gather(ut_v, [s16])
            dt = plsc.load_gather(ut_v, [d16])
            sa = st >= UNIT_BOUNDARY
            da = dt >= UNIT_BOUNDARY
            inr = (base + j * 16 + lanes) < E
            cross = sa & (~da) & inr
            mono = (~sa) & (~da) & inr
            hi = d16 >= HALF
            dloc = jnp.where(hi, d16 - HALF, d16)
            gr16 = r16 * NP + s16
            m_a0 = cross & (~hi)
            m_a1 = cross & hi
            m_r0 = mono & (~hi)
            m_r1 = mono & hi
            plsc.store_compressed(ga0_v.at[pl.ds(oa0, 16)], s16, mask=m_a0)
            plsc.store_compressed(da0_v.at[pl.ds(oa0, 16)], dloc, mask=m_a0)
            plsc.store_compressed(ga1_v.at[pl.ds(oa1, 16)], s16, mask=m_a1)
            plsc.store_compressed(da1_v.at[pl.ds(oa1, 16)], dloc, mask=m_a1)
            plsc.store_compressed(gr0_v.at[pl.ds(or0, 16)], gr16, mask=m_r0)
            plsc.store_compressed(dr0_v.at[pl.ds(or0, 16)], dloc, mask=m_r0)
            plsc.store_compressed(gr1_v.at[pl.ds(or1, 16)], gr16, mask=m_r1)
            plsc.store_compressed(dr1_v.at[pl.ds(or1, 16)], dloc, mask=m_r1)
            return (oa0 + jnp.sum(m_a0.astype(jnp.int32)),
                    oa1 + jnp.sum(m_a1.astype(jnp.int32)),
                    or0 + jnp.sum(m_r0.astype(jnp.int32)),
                    or1 + jnp.sum(m_r1.astype(jnp.int32)))

        return lax.fori_loop(0, CH // 16, grp, offs)

    z = jnp.int32(0)
    offs = lax.fori_loop(0, EPT // CH, chunk, (z, z, z, z))

    def pad(gref, dref, off):
        gbase = (off // 16) * 16

        def pgrp(i, _):
            idx = gbase + i * 16 + lanes
            m = idx >= off
            plsc.store_scatter(gref, [idx], zeros16, mask=m)
            plsc.store_scatter(dref, [idx], trash16, mask=m)
            return 0

        lax.fori_loop(0, 9, pgrp, 0)

    pad(ga0_v, da0_v, offs[0])
    pad(ga1_v, da1_v, offs[1])
    pad(gr0_v, dr0_v, offs[2])
    pad(gr1_v, dr1_v, offs[3])

    for cc, (gav, dav, grv, drv) in enumerate(
            ((ga0_v, da0_v, gr0_v, dr0_v), (ga1_v, da1_v, gr1_v, dr1_v))):
        pltpu.sync_copy(gav.at[pl.ds(0, EPT)], ga_h.at[cc, t])
        pltpu.sync_copy(dav.at[pl.ds(0, EPT)], da_h.at[cc, t])
        pltpu.sync_copy(grv.at[pl.ds(0, EPT)], gr_h.at[cc, t])
        pltpu.sync_copy(drv.at[pl.ds(0, EPT)], dr_h.at[cc, t])

    cntv[...] = jnp.where(lanes == 0, offs[0],
                          jnp.where(lanes == 8, offs[1], 0))
    pltpu.sync_copy(cntv.at[pl.ds(0, 8)], ca_h.at[0, pl.ds(t * 8, 8)])
    pltpu.sync_copy(cntv.at[pl.ds(8, 8)], ca_h.at[1, pl.ds(t * 8, 8)])
    cntv[...] = jnp.where(lanes == 0, offs[2],
                          jnp.where(lanes == 8, offs[3], 0))
    pltpu.sync_copy(cntv.at[pl.ds(0, 8)], cr_h.at[0, pl.ds(t * 8, 8)])
    pltpu.sync_copy(cntv.at[pl.ds(8, 8)], cr_h.at[1, pl.ds(t * 8, 8)])


def _sc_prep(src, dst, rel, ut):
    ei = jax.ShapeDtypeStruct((NC, TILES, EPT), jnp.int32)
    ci = jax.ShapeDtypeStruct((NC, TILES * 8), jnp.int32)
    fn = pl.kernel(
        _prep_body,
        out_type=(ei, ei, ei, ei, ci, ci),
        mesh=_mesh,
        compiler_params=pltpu.CompilerParams(needs_layout_passes=False),
        scratch_types=[
            pltpu.VMEM((NP,), jnp.int32),
            pltpu.VMEM((CH,), jnp.int32),
            pltpu.VMEM((CH,), jnp.int32),
            pltpu.VMEM((CH,), jnp.int32),
            pltpu.VMEM((16,), jnp.int32),
        ] + [pltpu.VMEM((CAP,), jnp.int32) for _ in range(8)],
    )
    return fn(src, dst, rel, ut)


def _scatter_body(table_h, g_h, d_h, cnt_h, out_h, gv, dv, cntv, rows_v, acc,
                  sem):
    c = lax.axis_index("c")
    s = lax.axis_index("s")

    def zlane(i, _):
        rows_v[0, i // 8, pl.ds((i % 8) * 16, 16)] = jnp.zeros((16,),
                                                              jnp.float32)
        return 0

    lax.fori_loop(0, K * 8, zlane, 0)

    zspan = ACC_R // NS

    def zcopy(k, _):
        pltpu.sync_copy(rows_v.at[0], acc.at[pl.ds(s * zspan + k * K, K), :])
        return 0

    lax.fori_loop(0, zspan // K, zcopy, 0)
    pltpu.sync_copy(rows_v.at[0].at[pl.ds(0, zspan % K)],
                    acc.at[pl.ds(s * zspan + (zspan // K) * K, zspan % K), :])
    plsc.subcore_barrier()

    for li in range(2):
        t = s * 2 + li
        pltpu.sync_copy(cnt_h.at[c, pl.ds(t * 8, 8)], cntv.at[pl.ds(0, 8)])
        n = cntv[pl.ds(0, 16)][0]
        nch = (n + (K - 1)) // K
        pltpu.sync_copy(g_h.at[c, t], gv)
        pltpu.sync_copy(d_h.at[c, t], dv)

        @pl.when(nch > 0)
        def _():
            pltpu.async_copy(table_h.at[gv.at[0]], rows_v.at[0], sem)

        def chunk(k, _):
            pltpu.make_async_copy(table_h.at[gv.at[k]], rows_v.at[k % 2],
                                  sem).wait()

            @pl.when(k + 1 < nch)
            def _():
                pltpu.async_copy(table_h.at[gv.at[k + 1]],
                                 rows_v.at[(k + 1) % 2], sem)

            pltpu.sync_copy(rows_v.at[k % 2], acc.at[dv.at[k]], add=True)
            return 0

        lax.fori_loop(0, nch, chunk, 0)

    plsc.subcore_barrier()
    span = HALF // NS

    def ocopy(k, _):
        r0 = s * span + k * 64
        pltpu.sync_copy(acc.at[pl.ds(r0, 64), :],
                        out_h.at[pl.ds(c * HALF + r0, 64), :])
        return 0

    lax.fori_loop(0, span // 64, ocopy, 0)


def _sc_scatter(table, gidx, didx, cnt):
    fn = pl.kernel(
        _scatter_body,
        out_type=jax.ShapeDtypeStruct((NP, D), jnp.float32),
        mesh=_mesh,
        scratch_types=[
            pltpu.VMEM((CPT, K), jnp.int32),
            pltpu.VMEM((CPT, K), jnp.int32),
            pltpu.VMEM((16,), jnp.int32),
            pltpu.VMEM((2, K, D), jnp.float32),
            pltpu.VMEM_SHARED((ACC_R, D), jnp.float32),
            pltpu.SemaphoreType.DMA,
        ],
    )
    return fn(table, gidx.reshape(NC, TILES, CPT, K),
              didx.reshape(NC, TILES, CPT, K), cnt)


def _k0_body(ut_ref, emb_ref, wrel_ref, x_ref, xw_ref):
    ut = ut_ref[0, 0, :]
    oh = (ut[:, None] ==
          lax.broadcasted_iota(jnp.int32, (BL, NUM_UNIT), 1)).astype(jnp.float32)
    x = _dot(oh, emb_ref[...])
    x_ref[...] = x
    for r in range(NUM_REL):
        xw_ref[r, :, :] = _dot(x, wrel_ref[r])


def _tc_k0(ut3, emb, wrel):
    return pl.pallas_call(
        _k0_body,
        grid=(NPB,),
        in_specs=[
            pl.BlockSpec((1, 1, BL), lambda i: (i, 0, 0)),
            pl.BlockSpec((NUM_UNIT, D), lambda i: (0, 0)),
            pl.BlockSpec((NUM_REL, D, D), lambda i: (0, 0, 0)),
        ],
        out_specs=[
            pl.BlockSpec((BL, D), lambda i: (i, 0)),
            pl.BlockSpec((NUM_REL, BL, D), lambda i: (0, i, 0)),
        ],
        out_shape=[
            jax.ShapeDtypeStruct((NP, D), jnp.float32),
            jax.ShapeDtypeStruct((NUM_REL, NP, D), jnp.float32),
        ],
    )(ut3, emb, wrel)


def _k2_body(agg_ref, li_in_ref, atom_ref, ws_ref, wa_ref, b_ref, wrel_ref,
             li_ref, xw_ref):
    li = jnp.maximum(
        agg_ref[...] + _dot(li_in_ref[...], ws_ref[...])
        + _dot(atom_ref[...], wa_ref[...]) + b_ref[...], 0.0)
    li_ref[...] = li
    for r in range(NUM_REL):
        xw_ref[r, :, :] = _dot(li, wrel_ref[r])


def _tc_k2(agg, li_prev, atom, ws, wa, b, wrel):
    return pl.pallas_call(
        _k2_body,
        grid=(NPB,),
        in_specs=[
            pl.BlockSpec((BL, D), lambda i: (i, 0)),
            pl.BlockSpec((BL, D), lambda i: (i, 0)),
            pl.BlockSpec((BL, D), lambda i: (i, 0)),
            pl.BlockSpec((D, D), lambda i: (0, 0)),
            pl.BlockSpec((D, D), lambda i: (0, 0)),
            pl.BlockSpec((1, D), lambda i: (0, 0)),
            pl.BlockSpec((NUM_REL, D, D), lambda i: (0, 0, 0)),
        ],
        out_specs=[
            pl.BlockSpec((BL, D), lambda i: (i, 0)),
            pl.BlockSpec((NUM_REL, BL, D), lambda i: (0, i, 0)),
        ],
        out_shape=[
            jax.ShapeDtypeStruct((NP, D), jnp.float32),
            jax.ShapeDtypeStruct((NUM_REL, NP, D), jnp.float32),
        ],
    )(agg, li_prev, atom, ws, wa, b, wrel)


def _k3_body(agg_ref, li_in_ref, atom_ref, ws_ref, wa_ref, b_ref,
             ut_ref, n2g_ref, nf_ref, gf_ref):
    li = jnp.maximum(
        agg_ref[...] + _dot(li_in_ref[...], ws_ref[...])
        + _dot(atom_ref[...], wa_ref[...]) + b_ref[...], 0.0)
    mono = (ut_ref[0, 0, :] < UNIT_BOUNDARY).astype(jnp.float32)
    nf = li * mono[:, None]
    nf_ref[...] = nf
    seg = n2g_ref[0, 0, :]
    oh = (lax.broadcasted_iota(jnp.int32, (B, BL), 0) ==
          seg[None, :]).astype(jnp.float32)

    @pl.when(pl.program_id(0) == 0)
    def _():
        gf_ref[...] = jnp.zeros((B, D), jnp.float32)

    gf_ref[...] += _dot(oh, nf)


def _tc_k3(agg, li_prev, atom, ws, wa, b, ut3, n2g3):
    return pl.pallas_call(
        _k3_body,
        grid=(NPB,),
        in_specs=[
            pl.BlockSpec((BL, D), lambda i: (i, 0)),
            pl.BlockSpec((BL, D), lambda i: (i, 0)),
            pl.BlockSpec((BL, D), lambda i: (i, 0)),
            pl.BlockSpec((D, D), lambda i: (0, 0)),
            pl.BlockSpec((D, D), lambda i: (0, 0)),
            pl.BlockSpec((1, D), lambda i: (0, 0)),
            pl.BlockSpec((1, 1, BL), lambda i: (i, 0, 0)),
            pl.BlockSpec((1, 1, BL), lambda i: (i, 0, 0)),
        ],
        out_specs=[
            pl.BlockSpec((BL, D), lambda i: (i, 0)),
            pl.BlockSpec((B, D), lambda i: (0, 0)),
        ],
        out_shape=[
            jax.ShapeDtypeStruct((NP, D), jnp.float32),
            jax.ShapeDtypeStruct((B, D), jnp.float32),
        ],
        compiler_params=pltpu.CompilerParams(
            dimension_semantics=("arbitrary",)),
    )(agg, li_prev, atom, ws, wa, b, ut3, n2g3)


def kernel(input, embedding, W_rel, W_self, W_atom, bias, unit_type,
           edge_index, edge_relation, node2graph):
    del input
    i32 = jnp.int32
    ut = unit_type.astype(i32)
    utp = jnp.concatenate([ut, jnp.full((NP - N,), NUM_UNIT - 1, i32)])
    ut3 = utp.reshape(NPB, 1, BL)
    src = jnp.concatenate([edge_index[0].astype(i32),
                           jnp.zeros((EP - E,), i32)])
    dst = jnp.concatenate([edge_index[1].astype(i32),
                           jnp.zeros((EP - E,), i32)])
    rel = jnp.concatenate([edge_relation.astype(i32),
                           jnp.zeros((EP - E,), i32)])
    n2g = jnp.concatenate([node2graph.astype(i32), jnp.zeros((NP - N,), i32)])
    n2g3 = n2g.reshape(NPB, 1, BL)
    bias2 = bias.reshape(NUM_BLOCKS, 1, D)

    x, xw0 = _tc_k0(ut3, embedding, W_rel[0])
    ga, da, gr, dr, ca, cr = _sc_prep(src, dst, rel, utp)
    atom = _sc_scatter(x, ga, da, ca)
    agg0 = _sc_scatter(xw0.reshape(NUM_REL * NP, D), gr, dr, cr)
    li1, xw1 = _tc_k2(agg0, x, atom, W_self[0], W_atom[0], bias2[0], W_rel[1])
    agg1 = _sc_scatter(xw1.reshape(NUM_REL * NP, D), gr, dr, cr)
    li2, xw2 = _tc_k2(agg1, li1, atom, W_self[1], W_atom[1], bias2[1],
                      W_rel[2])
    agg2 = _sc_scatter(xw2.reshape(NUM_REL * NP, D), gr, dr, cr)
    nf, gf = _tc_k3(agg2, li2, atom, W_self[2], W_atom[2], bias2[2],
                    ut3, n2g3)
    return gf, nf[:N]


# speedup vs baseline: 17.4899x; 1.0164x over previous
import jax
import jax.numpy as jnp
from jax import lax
from jax.experimental import pallas as pl
from jax.experimental.pallas import tpu as pltpu
import jax.experimental.pallas.tpu_sc as plsc

N = 10000
E = 320000
D = 128
NUM_UNIT = 512
UNIT_BOUNDARY = 256
NUM_REL = 4
NUM_BLOCKS = 3
B = 64

NP = 10240
BL = 1024
NPB = NP // BL
NC = 2
NS = 16
TILES = NC * NS
K = 128
CPT = 80
EPT = CPT * K
EP = TILES * EPT
HALF = NP // 2
ACC_R = HALF + NS
CAP = EPT + 256
CH = 2048

_HIGH = jax.lax.Precision.HIGHEST


def _dot(a, b):
    return jnp.dot(a, b, precision=_HIGH, preferred_element_type=jnp.float32)


_mesh = plsc.VectorSubcoreMesh(core_axis_name="c", subcore_axis_name="s")


def _prep_body(src_h, dst_h, rel_h, ut_h,
               ga_h, da_h, gr_h, dr_h, ca_h, cr_h,
               ut_v, sv, dv, rv, cntv,
               ga0_v, ga1_v, da0_v, da1_v, gr0_v, gr1_v, dr0_v, dr1_v):
    c = lax.axis_index("c")
    s = lax.axis_index("s")
    t = s * NC + c
    trash = jnp.int32(HALF + (t // NC))
    pltpu.sync_copy(ut_h, ut_v)
    lanes = lax.broadcasted_iota(jnp.int32, (16,), 0)
    zeros16 = jnp.zeros((16,), jnp.int32)
    trash16 = jnp.full((16,), trash, jnp.int32)

    def chunk(ch, offs):
        base = t * EPT + ch * CH
        pltpu.sync_copy(src_h.at[pl.ds(base, CH)], sv)
        pltpu.sync_copy(dst_h.at[pl.ds(base, CH)], dv)
        pltpu.sync_copy(rel_h.at[pl.ds(base, CH)], rv)

        def grp(j, offs):
            oa0, oa1, or0, or1 = offs
            s16 = sv[pl.ds(j * 16, 16)]
            d16 = dv[pl.ds(j * 16, 16)]
            r16 = rv[pl.ds(j * 16, 16)]
            st = plsc.load_gather(ut_v, [s16])
            dt = plsc.load_gather(ut_v, [d16])
            sa = st >= UNIT_BOUNDARY
            da = dt >= UNIT_BOUNDARY
            inr = (base + j * 16 + lanes) < E
            cross = sa & (~da) & inr
            mono = (~sa) & (~da) & inr
            hi = d16 >= HALF
            dloc = jnp.where(hi, d16 - HALF, d16)
            gr16 = r16 * NP + s16
            m_a0 = cross & (~hi)
            m_a1 = cross & hi
            m_r0 = mono & (~hi)
            m_r1 = mono & hi
            plsc.store_compressed(ga0_v.at[pl.ds(oa0, 16)], s16, mask=m_a0)
            plsc.store_compressed(da0_v.at[pl.ds(oa0, 16)], dloc, mask=m_a0)
            plsc.store_compressed(ga1_v.at[pl.ds(oa1, 16)], s16, mask=m_a1)
            plsc.store_compressed(da1_v.at[pl.ds(oa1, 16)], dloc, mask=m_a1)
            plsc.store_compressed(gr0_v.at[pl.ds(or0, 16)], gr16, mask=m_r0)
            plsc.store_compressed(dr0_v.at[pl.ds(or0, 16)], dloc, mask=m_r0)
            plsc.store_compressed(gr1_v.at[pl.ds(or1, 16)], gr16, mask=m_r1)
            plsc.store_compressed(dr1_v.at[pl.ds(or1, 16)], dloc, mask=m_r1)
            return (oa0 + jnp.sum(m_a0.astype(jnp.int32)),
                    oa1 + jnp.sum(m_a1.astype(jnp.int32)),
                    or0 + jnp.sum(m_r0.astype(jnp.int32)),
                    or1 + jnp.sum(m_r1.astype(jnp.int32)))

        return lax.fori_loop(0, CH // 16, grp, offs)

    z = jnp.int32(0)
    offs = lax.fori_loop(0, EPT // CH, chunk, (z, z, z, z))

    def pad(gref, dref, off):
        gbase = (off // 16) * 16

        def pgrp(i, _):
            idx = gbase + i * 16 + lanes
            m = idx >= off
            plsc.store_scatter(gref, [idx], zeros16, mask=m)
            plsc.store_scatter(dref, [idx], trash16, mask=m)
            return 0

        lax.fori_loop(0, 9, pgrp, 0)

    pad(ga0_v, da0_v, offs[0])
    pad(ga1_v, da1_v, offs[1])
    pad(gr0_v, dr0_v, offs[2])
    pad(gr1_v, dr1_v, offs[3])

    for cc, (gav, dav, grv, drv) in enumerate(
            ((ga0_v, da0_v, gr0_v, dr0_v), (ga1_v, da1_v, gr1_v, dr1_v))):
        pltpu.sync_copy(gav.at[pl.ds(0, EPT)], ga_h.at[cc, t])
        pltpu.sync_copy(dav.at[pl.ds(0, EPT)], da_h.at[cc, t])
        pltpu.sync_copy(grv.at[pl.ds(0, EPT)], gr_h.at[cc, t])
        pltpu.sync_copy(drv.at[pl.ds(0, EPT)], dr_h.at[cc, t])

    cntv[...] = jnp.where(lanes == 0, offs[0],
                          jnp.where(lanes == 8, offs[1], 0))
    pltpu.sync_copy(cntv.at[pl.ds(0, 8)], ca_h.at[0, pl.ds(t * 8, 8)])
    pltpu.sync_copy(cntv.at[pl.ds(8, 8)], ca_h.at[1, pl.ds(t * 8, 8)])
    cntv[...] = jnp.where(lanes == 0, offs[2],
                          jnp.where(lanes == 8, offs[3], 0))
    pltpu.sync_copy(cntv.at[pl.ds(0, 8)], cr_h.at[0, pl.ds(t * 8, 8)])
    pltpu.sync_copy(cntv.at[pl.ds(8, 8)], cr_h.at[1, pl.ds(t * 8, 8)])


def _sc_prep(src, dst, rel, ut):
    ei = jax.ShapeDtypeStruct((NC, TILES, EPT), jnp.int32)
    ci = jax.ShapeDtypeStruct((NC, TILES * 8), jnp.int32)
    fn = pl.kernel(
        _prep_body,
        out_type=(ei, ei, ei, ei, ci, ci),
        mesh=_mesh,
        compiler_params=pltpu.CompilerParams(needs_layout_passes=False),
        scratch_types=[
            pltpu.VMEM((NP,), jnp.int32),
            pltpu.VMEM((CH,), jnp.int32),
            pltpu.VMEM((CH,), jnp.int32),
            pltpu.VMEM((CH,), jnp.int32),
            pltpu.VMEM((16,), jnp.int32),
        ] + [pltpu.VMEM((CAP,), jnp.int32) for _ in range(8)],
    )
    return fn(src, dst, rel, ut)


def _scatter_body(table_h, g_h, d_h, cnt_h, out_h, gv, dv, cntv, rows_v, acc,
                  sem, ssem):
    c = lax.axis_index("c")
    s = lax.axis_index("s")

    def zlane(i, _):
        rows_v[0, i // 8, pl.ds((i % 8) * 16, 16)] = jnp.zeros((16,),
                                                              jnp.float32)
        return 0

    lax.fori_loop(0, K * 8, zlane, 0)

    zspan = ACC_R // NS

    def zcopy(k, _):
        pltpu.sync_copy(rows_v.at[0], acc.at[pl.ds(s * zspan + k * K, K), :])
        return 0

    lax.fori_loop(0, zspan // K, zcopy, 0)
    pltpu.sync_copy(rows_v.at[0].at[pl.ds(0, zspan % K)],
                    acc.at[pl.ds(s * zspan + (zspan // K) * K, zspan % K), :])
    plsc.subcore_barrier()

    for li in range(2):
        t = s * 2 + li
        pltpu.sync_copy(cnt_h.at[c, pl.ds(t * 8, 8)], cntv.at[pl.ds(0, 8)])
        n = cntv[pl.ds(0, 16)][0]
        nch = (n + (K - 1)) // K
        pltpu.sync_copy(g_h.at[c, t], gv)
        pltpu.sync_copy(d_h.at[c, t], dv)

        def gissue(k):
            pltpu.async_copy(table_h.at[gv.at[k]], rows_v.at[k % 3], sem)

        def gwait(k):
            pltpu.make_async_copy(table_h.at[gv.at[k]], rows_v.at[k % 3],
                                  sem).wait()

        def sissue(k):
            pltpu.async_copy(rows_v.at[k % 3], acc.at[dv.at[k]], ssem,
                             add=True)

        def swait(k):
            pltpu.make_async_copy(rows_v.at[k % 3], acc.at[dv.at[k]],
                                  ssem).wait()

        @pl.when(nch > 0)
        def _():
            gissue(0)

        @pl.when(nch > 1)
        def _():
            gissue(1)

        def chunk(k, _):
            gwait(k)
            sissue(k)

            @pl.when(k + 2 < nch)
            def _():
                @pl.when(k >= 1)
                def _():
                    swait(k - 1)

                gissue(k + 2)

            return 0

        lax.fori_loop(0, nch, chunk, 0)
        for dback in range(3, 0, -1):
            @pl.when(nch >= dback)
            def _(dback=dback):
                swait(nch - dback)

    plsc.subcore_barrier()
    span = HALF // NS

    def ocopy(k, _):
        r0 = s * span + k * 64
        pltpu.sync_copy(acc.at[pl.ds(r0, 64), :],
                        out_h.at[pl.ds(c * HALF + r0, 64), :])
        return 0

    lax.fori_loop(0, span // 64, ocopy, 0)


def _sc_scatter(table, gidx, didx, cnt):
    fn = pl.kernel(
        _scatter_body,
        out_type=jax.ShapeDtypeStruct((NP, D), jnp.float32),
        mesh=_mesh,
        scratch_types=[
            pltpu.VMEM((CPT, K), jnp.int32),
            pltpu.VMEM((CPT, K), jnp.int32),
            pltpu.VMEM((16,), jnp.int32),
            pltpu.VMEM((3, K, D), jnp.float32),
            pltpu.VMEM_SHARED((ACC_R, D), jnp.float32),
            pltpu.SemaphoreType.DMA,
            pltpu.SemaphoreType.DMA,
        ],
    )
    return fn(table, gidx.reshape(NC, TILES, CPT, K),
              didx.reshape(NC, TILES, CPT, K), cnt)


def _k0_body(ut_ref, emb_ref, wrel_ref, x_ref, xw_ref):
    ut = ut_ref[0, 0, :]
    oh = (ut[:, None] ==
          lax.broadcasted_iota(jnp.int32, (BL, NUM_UNIT), 1)).astype(jnp.float32)
    x = _dot(oh, emb_ref[...])
    x_ref[...] = x
    for r in range(NUM_REL):
        xw_ref[r, :, :] = _dot(x, wrel_ref[r])


def _tc_k0(ut3, emb, wrel):
    return pl.pallas_call(
        _k0_body,
        grid=(NPB,),
        in_specs=[
            pl.BlockSpec((1, 1, BL), lambda i: (i, 0, 0)),
            pl.BlockSpec((NUM_UNIT, D), lambda i: (0, 0)),
            pl.BlockSpec((NUM_REL, D, D), lambda i: (0, 0, 0)),
        ],
        out_specs=[
            pl.BlockSpec((BL, D), lambda i: (i, 0)),
            pl.BlockSpec((NUM_REL, BL, D), lambda i: (0, i, 0)),
        ],
        out_shape=[
            jax.ShapeDtypeStruct((NP, D), jnp.float32),
            jax.ShapeDtypeStruct((NUM_REL, NP, D), jnp.float32),
        ],
    )(ut3, emb, wrel)


def _k2_body(agg_ref, li_in_ref, atom_ref, ws_ref, wa_ref, b_ref, wrel_ref,
             li_ref, xw_ref):
    li = jnp.maximum(
        agg_ref[...] + _dot(li_in_ref[...], ws_ref[...])
        + _dot(atom_ref[...], wa_ref[...]) + b_ref[...], 0.0)
    li_ref[...] = li
    for r in range(NUM_REL):
        xw_ref[r, :, :] = _dot(li, wrel_ref[r])


def _tc_k2(agg, li_prev, atom, ws, wa, b, wrel):
    return pl.pallas_call(
        _k2_body,
        grid=(NPB,),
        in_specs=[
            pl.BlockSpec((BL, D), lambda i: (i, 0)),
            pl.BlockSpec((BL, D), lambda i: (i, 0)),
            pl.BlockSpec((BL, D), lambda i: (i, 0)),
            pl.BlockSpec((D, D), lambda i: (0, 0)),
            pl.BlockSpec((D, D), lambda i: (0, 0)),
            pl.BlockSpec((1, D), lambda i: (0, 0)),
            pl.BlockSpec((NUM_REL, D, D), lambda i: (0, 0, 0)),
        ],
        out_specs=[
            pl.BlockSpec((BL, D), lambda i: (i, 0)),
            pl.BlockSpec((NUM_REL, BL, D), lambda i: (0, i, 0)),
        ],
        out_shape=[
            jax.ShapeDtypeStruct((NP, D), jnp.float32),
            jax.ShapeDtypeStruct((NUM_REL, NP, D), jnp.float32),
        ],
    )(agg, li_prev, atom, ws, wa, b, wrel)


def _k3_body(agg_ref, li_in_ref, atom_ref, ws_ref, wa_ref, b_ref,
             ut_ref, n2g_ref, nf_ref, gf_ref):
    li = jnp.maximum(
        agg_ref[...] + _dot(li_in_ref[...], ws_ref[...])
        + _dot(atom_ref[...], wa_ref[...]) + b_ref[...], 0.0)
    mono = (ut_ref[0, 0, :] < UNIT_BOUNDARY).astype(jnp.float32)
    nf = li * mono[:, None]
    nf_ref[...] = nf
    seg = n2g_ref[0, 0, :]
    oh = (lax.broadcasted_iota(jnp.int32, (B, BL), 0) ==
          seg[None, :]).astype(jnp.float32)

    @pl.when(pl.program_id(0) == 0)
    def _():
        gf_ref[...] = jnp.zeros((B, D), jnp.float32)

    gf_ref[...] += _dot(oh, nf)


def _tc_k3(agg, li_prev, atom, ws, wa, b, ut3, n2g3):
    return pl.pallas_call(
        _k3_body,
        grid=(NPB,),
        in_specs=[
            pl.BlockSpec((BL, D), lambda i: (i, 0)),
            pl.BlockSpec((BL, D), lambda i: (i, 0)),
            pl.BlockSpec((BL, D), lambda i: (i, 0)),
            pl.BlockSpec((D, D), lambda i: (0, 0)),
            pl.BlockSpec((D, D), lambda i: (0, 0)),
            pl.BlockSpec((1, D), lambda i: (0, 0)),
            pl.BlockSpec((1, 1, BL), lambda i: (i, 0, 0)),
            pl.BlockSpec((1, 1, BL), lambda i: (i, 0, 0)),
        ],
        out_specs=[
            pl.BlockSpec((BL, D), lambda i: (i, 0)),
            pl.BlockSpec((B, D), lambda i: (0, 0)),
        ],
        out_shape=[
            jax.ShapeDtypeStruct((NP, D), jnp.float32),
            jax.ShapeDtypeStruct((B, D), jnp.float32),
        ],
        compiler_params=pltpu.CompilerParams(
            dimension_semantics=("arbitrary",)),
    )(agg, li_prev, atom, ws, wa, b, ut3, n2g3)


def kernel(input, embedding, W_rel, W_self, W_atom, bias, unit_type,
           edge_index, edge_relation, node2graph):
    del input
    i32 = jnp.int32
    ut = unit_type.astype(i32)
    utp = jnp.concatenate([ut, jnp.full((NP - N,), NUM_UNIT - 1, i32)])
    ut3 = utp.reshape(NPB, 1, BL)
    src = jnp.concatenate([edge_index[0].astype(i32),
                           jnp.zeros((EP - E,), i32)])
    dst = jnp.concatenate([edge_index[1].astype(i32),
                           jnp.zeros((EP - E,), i32)])
    rel = jnp.concatenate([edge_relation.astype(i32),
                           jnp.zeros((EP - E,), i32)])
    n2g = jnp.concatenate([node2graph.astype(i32), jnp.zeros((NP - N,), i32)])
    n2g3 = n2g.reshape(NPB, 1, BL)
    bias2 = bias.reshape(NUM_BLOCKS, 1, D)

    x, xw0 = _tc_k0(ut3, embedding, W_rel[0])
    ga, da, gr, dr, ca, cr = _sc_prep(src, dst, rel, utp)
    atom = _sc_scatter(x, ga, da, ca)
    agg0 = _sc_scatter(xw0.reshape(NUM_REL * NP, D), gr, dr, cr)
    li1, xw1 = _tc_k2(agg0, x, atom, W_self[0], W_atom[0], bias2[0], W_rel[1])
    agg1 = _sc_scatter(xw1.reshape(NUM_REL * NP, D), gr, dr, cr)
    li2, xw2 = _tc_k2(agg1, li1, atom, W_self[1], W_atom[1], bias2[1],
                      W_rel[2])
    agg2 = _sc_scatter(xw2.reshape(NUM_REL * NP, D), gr, dr, cr)
    nf, gf = _tc_k3(agg2, li2, atom, W_self[2], W_atom[2], bias2[2],
                    ut3, n2g3)
    return gf, nf[:N]
